# baseline (device time: 155021 ns/iter reference)
import jax
import jax.numpy as jnp
from jax import lax
from jax.experimental import pallas as pl
from jax.experimental.pallas import tpu as pltpu

N_DEV = 16
E_LOCAL = 4
C = 96
PAY = 384


def kernel(x, router_W, route_idx, expert_W, shared_W):
    T, D = x.shape
    H = shared_W.shape[1]

    g = route_idx[:, 0]
    dest = g // E_LOCAL
    e_loc = g % E_LOCAL
    order = jnp.argsort(dest)
    sd = dest[order]
    rank = jnp.arange(T) - jnp.searchsorted(sd, sd)
    payload = jnp.zeros((T, PAY), jnp.bfloat16)
    payload = payload.at[:, :D].set(x.astype(jnp.bfloat16))
    payload = payload.at[:, D].set(e_loc.astype(jnp.bfloat16))
    xd = (
        jnp.zeros((N_DEV, C, PAY), jnp.bfloat16)
        .at[sd, rank].set(payload[order], mode="drop")
    )
    slot = jnp.zeros((T,), jnp.int32).at[order].set(
        (sd * C + rank).astype(jnp.int32)
    )

    probs = jax.nn.softmax(x @ router_W, axis=-1)
    p = jnp.take_along_axis(probs, route_idx, axis=1)[:, 0]

    def body(xd_ref, ew_ref, x_ref, sw_ref, y_out, sh_out,
             xr_ref, ys_ref, send_x, recv_x, send_y, recv_y):
        my = lax.axis_index("i")

        def dispatch_desc(d):
            return pltpu.make_async_remote_copy(
                src_ref=xd_ref.at[d], dst_ref=xr_ref.at[my],
                send_sem=send_x.at[d], recv_sem=recv_x.at[my],
                device_id=(d,), device_id_type=pl.DeviceIdType.MESH,
            )

        def recv_x_desc(s):
            return pltpu.make_async_remote_copy(
                src_ref=xd_ref.at[s], dst_ref=xr_ref.at[s],
                send_sem=send_x.at[s], recv_sem=recv_x.at[s],
                device_id=(s,), device_id_type=pl.DeviceIdType.MESH,
            )

        def return_desc(s):
            return pltpu.make_async_remote_copy(
                src_ref=ys_ref.at[s], dst_ref=y_out.at[my],
                send_sem=send_y.at[s], recv_sem=recv_y.at[my],
                device_id=(s,), device_id_type=pl.DeviceIdType.MESH,
            )

        def recv_y_desc(s):
            return pltpu.make_async_remote_copy(
                src_ref=ys_ref.at[s], dst_ref=y_out.at[s],
                send_sem=send_y.at[s], recv_sem=recv_y.at[s],
                device_id=(s,), device_id_type=pl.DeviceIdType.MESH,
            )

        barrier_sem = pltpu.get_barrier_semaphore()
        for d in range(N_DEV):
            @pl.when(d != my)
            def _(d=d):
                pl.semaphore_signal(
                    barrier_sem, inc=1,
                    device_id=(d,), device_id_type=pl.DeviceIdType.MESH,
                )
        pl.semaphore_wait(barrier_sem, N_DEV - 1)

        for d in range(N_DEV):
            @pl.when(d != my)
            def _(d=d):
                dispatch_desc(d).start()

        xr_ref[pl.ds(my, 1)] = xd_ref[pl.ds(my, 1)]

        xb = x_ref[:, :].astype(jnp.bfloat16)
        sh_out[:, :] = jnp.dot(xb, sw_ref[:, :].astype(jnp.bfloat16),
                               preferred_element_type=jnp.float32)

        w = ew_ref[:, :, :].reshape(E_LOCAL * D, H).astype(jnp.bfloat16)

        for s in range(N_DEV):
            @pl.when(s != my)
            def _(s=s):
                recv_x_desc(s).wait_recv()
            xr = xr_ref[s]
            xtok = xr[:, :D]
            el = xr[:, D:D + 1]
            cols = [
                jnp.where(el == e, xtok, jnp.bfloat16(0.0))
                for e in range(E_LOCAL)
            ]
            xg = jnp.concatenate(cols, axis=1)
            yr = jnp.dot(xg, w, preferred_element_type=jnp.float32)
            ys_ref[s] = yr.astype(jnp.bfloat16)

            @pl.when(s != my)
            def _(s=s):
                return_desc(s).start()

            @pl.when(s == my)
            def _(s=s):
                y_out[s] = ys_ref[s]

        for s in range(N_DEV):
            @pl.when(s != my)
            def _(s=s):
                dispatch_desc(s).wait_send()
                return_desc(s).wait_send()
                recv_y_desc(s).wait_recv()

    y_all, sh = pl.pallas_call(
        body,
        out_shape=[
            jax.ShapeDtypeStruct((N_DEV, C, H), jnp.bfloat16),
            jax.ShapeDtypeStruct((T, H), jnp.float32),
        ],
        in_specs=[pl.BlockSpec(memory_space=pltpu.VMEM)] * 4,
        out_specs=[pl.BlockSpec(memory_space=pltpu.VMEM)] * 2,
        scratch_shapes=[
            pltpu.VMEM((N_DEV, C, PAY), jnp.bfloat16),
            pltpu.VMEM((N_DEV, C, H), jnp.bfloat16),
            pltpu.SemaphoreType.DMA((N_DEV,)),
            pltpu.SemaphoreType.DMA((N_DEV,)),
            pltpu.SemaphoreType.DMA((N_DEV,)),
            pltpu.SemaphoreType.DMA((N_DEV,)),
        ],
        compiler_params=pltpu.CompilerParams(collective_id=0),
    )(xd, expert_W, x, shared_W)

    yflat = y_all.reshape(N_DEV * C, H).astype(jnp.float32)
    return sh + p[:, None] * yflat[slot]


# device time: 42913 ns/iter; 3.6124x vs baseline; 3.6124x over previous
import jax
import jax.numpy as jnp
from jax import lax
from jax.experimental import pallas as pl
from jax.experimental.pallas import tpu as pltpu

N_DEV = 16
E_LOCAL = 4
N_EXPERTS = N_DEV * E_LOCAL
C = 96
PAY = 384
S = N_DEV * C


def kernel(x, router_W, route_idx, expert_W, shared_W):
    T, D = x.shape
    H = shared_W.shape[1]

    def body(x_ref, router_ref, idx_ref, ew_ref, sw_ref, out_ref,
             xd_ref, xr_ref, ys_ref, yr_ref,
             send_x, recv_x, send_y, recv_y):
        my = lax.axis_index("i")

        def dispatch_desc(d):
            return pltpu.make_async_remote_copy(
                src_ref=xd_ref.at[d], dst_ref=xr_ref.at[my],
                send_sem=send_x.at[d], recv_sem=recv_x.at[my],
                device_id=(d,), device_id_type=pl.DeviceIdType.MESH,
            )

        def recv_x_desc(s):
            return pltpu.make_async_remote_copy(
                src_ref=xd_ref.at[s], dst_ref=xr_ref.at[s],
                send_sem=send_x.at[s], recv_sem=recv_x.at[s],
                device_id=(s,), device_id_type=pl.DeviceIdType.MESH,
            )

        def return_desc(s):
            return pltpu.make_async_remote_copy(
                src_ref=ys_ref.at[s], dst_ref=yr_ref.at[my],
                send_sem=send_y.at[s], recv_sem=recv_y.at[my],
                device_id=(s,), device_id_type=pl.DeviceIdType.MESH,
            )

        def recv_y_desc(s):
            return pltpu.make_async_remote_copy(
                src_ref=ys_ref.at[s], dst_ref=yr_ref.at[s],
                send_sem=send_y.at[s], recv_sem=recv_y.at[s],
                device_id=(s,), device_id_type=pl.DeviceIdType.MESH,
            )

        barrier_sem = pltpu.get_barrier_semaphore()
        for d in range(N_DEV):
            @pl.when(d != my)
            def _(d=d):
                pl.semaphore_signal(
                    barrier_sem, inc=1,
                    device_id=(d,), device_id_type=pl.DeviceIdType.MESH,
                )
        pl.semaphore_wait(barrier_sem, N_DEV - 1)

        g = idx_ref[:, :]
        dest = g // E_LOCAL
        e_loc = g % E_LOCAL
        xb = x_ref[:, :].astype(jnp.bfloat16)

        pad = jnp.zeros((T, PAY - D - 1), jnp.bfloat16)
        payload = jnp.concatenate(
            [xb, e_loc.astype(jnp.bfloat16), pad], axis=1)

        ohm = lax.broadcasted_iota(jnp.int32, (T, N_DEV), 1) == dest
        oh32 = jnp.where(ohm, jnp.float32(1.0), jnp.float32(0.0))
        tri = lax.broadcasted_iota(jnp.int32, (T, T), 0) > \
            lax.broadcasted_iota(jnp.int32, (T, T), 1)
        L = jnp.where(tri, jnp.float32(1.0),
                      jnp.float32(0.0)).astype(jnp.bfloat16)
        cnt = jnp.dot(L, oh32.astype(jnp.bfloat16),
                      preferred_element_type=jnp.float32)
        rank = jnp.sum(oh32 * cnt, axis=1,
                       keepdims=True).astype(jnp.int32)
        slot = dest * C + rank
        Pm = lax.broadcasted_iota(jnp.int32, (T, S), 1) == slot
        P = jnp.where(Pm, jnp.float32(1.0),
                      jnp.float32(0.0)).astype(jnp.bfloat16)

        xd = lax.dot_general(
            P, payload, (((0,), (0,)), ((), ())),
            preferred_element_type=jnp.float32,
        ).astype(jnp.bfloat16)
        xd_ref[:, :, :] = xd.reshape(N_DEV, C, PAY)

        for d in range(N_DEV):
            @pl.when(d != my)
            def _(d=d):
                dispatch_desc(d).start()

        xr_ref[pl.ds(my, 1)] = xd_ref[pl.ds(my, 1)]

        sh = jnp.dot(xb, sw_ref[:, :].astype(jnp.bfloat16),
                     preferred_element_type=jnp.float32)
        scores = jnp.dot(x_ref[:, :], router_ref[:, :],
                         preferred_element_type=jnp.float32)
        m = jnp.max(scores, axis=-1, keepdims=True)
        ex = jnp.exp(scores - m)
        probs = ex / jnp.sum(ex, axis=-1, keepdims=True)
        ohe = lax.broadcasted_iota(jnp.int32, (T, N_EXPERTS), 1) == g
        p = jnp.sum(jnp.where(ohe, probs, 0.0), axis=1,
                    keepdims=True)

        w = ew_ref[:, :, :].reshape(E_LOCAL * D, H).astype(jnp.bfloat16)

        for s in range(N_DEV):
            @pl.when(s != my)
            def _(s=s):
                recv_x_desc(s).wait_recv()
            xr = xr_ref[s]
            xtok = xr[:, :D]
            el = xr[:, D:D + 1]
            cols = [
                jnp.where(el == e, xtok, jnp.bfloat16(0.0))
                for e in range(E_LOCAL)
            ]
            xg = jnp.concatenate(cols, axis=1)
            yv = jnp.dot(xg, w, preferred_element_type=jnp.float32)
            ys_ref[s] = yv.astype(jnp.bfloat16)

            @pl.when(s != my)
            def _(s=s):
                return_desc(s).start()

            @pl.when(s == my)
            def _(s=s):
                yr_ref[s] = ys_ref[s]

        for s in range(N_DEV):
            @pl.when(s != my)
            def _(s=s):
                recv_y_desc(s).wait_recv()

        yflat = yr_ref[:, :, :].reshape(S, H)
        y_tok = jnp.dot(P, yflat, preferred_element_type=jnp.float32)
        out_ref[:, :] = sh + p * y_tok

        for s in range(N_DEV):
            @pl.when(s != my)
            def _(s=s):
                dispatch_desc(s).wait_send()
                return_desc(s).wait_send()

    return pl.pallas_call(
        body,
        out_shape=jax.ShapeDtypeStruct((T, H), jnp.float32),
        in_specs=[pl.BlockSpec(memory_space=pltpu.VMEM)] * 5,
        out_specs=pl.BlockSpec(memory_space=pltpu.VMEM),
        scratch_shapes=[
            pltpu.VMEM((N_DEV, C, PAY), jnp.bfloat16),
            pltpu.VMEM((N_DEV, C, PAY), jnp.bfloat16),
            pltpu.VMEM((N_DEV, C, H), jnp.bfloat16),
            pltpu.VMEM((N_DEV, C, H), jnp.bfloat16),
            pltpu.SemaphoreType.DMA((N_DEV,)),
            pltpu.SemaphoreType.DMA((N_DEV,)),
            pltpu.SemaphoreType.DMA((N_DEV,)),
            pltpu.SemaphoreType.DMA((N_DEV,)),
        ],
        compiler_params=pltpu.CompilerParams(collective_id=0),
    )(x, router_W, route_idx, expert_W, shared_W)
